# Initial kernel scaffold; baseline (speedup 1.0000x reference)
import jax
import jax.numpy as jnp
from jax.experimental import pallas as pl


def kernel(tensor, addresses, results):
    raise NotImplementedError("write your pallas kernel here")


# speedup vs baseline: 3.8531x; 3.8531x over previous
import jax
import jax.numpy as jnp
from jax import lax
from jax.experimental import pallas as pl
from jax.experimental.pallas import tpu as pltpu, tpu_sc as plsc

N = 1048576
W_IN = 8
W_OUT = 4
P = 16

NC = 2
NS = 16
NW = NC * NS
LANES = 16

ROWS_PER_W = N // NW
CH_ROWS = 4096
CH = CH_ROWS * W_IN
NCHUNK = ROWS_PER_W // CH_ROWS
LUT_SIZE = 256 * W_IN


def _vperm(x, idx):
    return lax.gather(
        x, idx[:, None],
        lax.GatherDimensionNumbers(
            offset_dims=(), collapsed_slice_dims=(0,), start_index_map=(0,)),
        slice_sizes=(1,),
        mode=lax.GatherScatterMode.PROMISE_IN_BOUNDS)


def _body(in_hbm, addr_hbm, res_hbm, out_hbm,
          buf0, buf1, lut, addr_v, res_v,
          in_sem0, in_sem1, out_sem0, out_sem1):
    wid = lax.axis_index("s") * NC + lax.axis_index("c")
    base = wid * (ROWS_PER_W * W_IN)

    pltpu.sync_copy(addr_hbm, addr_v)
    pltpu.sync_copy(res_hbm, res_v)

    lane = lax.iota(jnp.int32, LANES)
    laneoff = lane & (W_IN - 1)
    wpat = (jnp.int32(1) << laneoff).astype(jnp.float32)
    ix4 = lane ^ 4
    ix2 = lane ^ 2
    ix1 = lane ^ 1

    nanv = jnp.full((LANES,), jnp.nan, dtype=jnp.float32)

    def init_body(k, c):
        lut[pl.ds(k * LANES, LANES)] = nanv
        return c

    lax.fori_loop(0, LUT_SIZE // LANES, init_body, 0)

    code_p = addr_v[0, :]
    for j in range(1, W_IN):
        code_p = code_p + (addr_v[j, :] << j)
    for c in range(W_OUT):
        plsc.store_scatter(lut, [code_p * W_IN + c], res_v[c, :])

    def compute(buf):
        def body(k, c):
            off = k * LANES
            v = buf[pl.ds(off, LANES)]
            t = v * wpat
            s = t + _vperm(t, ix4)
            s = s + _vperm(s, ix2)
            s = s + _vperm(s, ix1)
            idx = s.astype(jnp.int32) * W_IN + laneoff
            e = plsc.load_gather(lut, [idx])
            buf[pl.ds(off, LANES)] = jnp.where(e == e, e, v)
            return c

        lax.fori_loop(0, CH // LANES, body, 0, unroll=8)

    bufs = (buf0, buf1)
    in_sems = (in_sem0, in_sem1)
    out_sems = (out_sem0, out_sem1)

    def in_copy(g):
        return pltpu.make_async_copy(
            in_hbm.at[pl.ds(base + g * CH, CH)], bufs[g % 2], in_sems[g % 2])

    def out_copy(g):
        return pltpu.make_async_copy(
            bufs[g % 2], out_hbm.at[pl.ds(base + g * CH, CH)], out_sems[g % 2])

    in_copy(0).start()
    for g in range(NCHUNK):
        if g + 1 < NCHUNK:
            if g >= 1:
                out_copy(g - 1).wait()
            in_copy(g + 1).start()
        in_copy(g).wait()
        compute(bufs[g % 2])
        out_copy(g).start()
    out_copy(NCHUNK - 2).wait()
    out_copy(NCHUNK - 1).wait()


@jax.jit
def _run(flat_in, addr_t, res_t):
    kfn = pl.kernel(
        _body,
        out_type=jax.ShapeDtypeStruct((N * W_IN,), jnp.float32),
        mesh=plsc.VectorSubcoreMesh(core_axis_name="c", subcore_axis_name="s"),
        compiler_params=pltpu.CompilerParams(needs_layout_passes=False),
        scratch_types=[
            pltpu.VMEM((CH,), jnp.float32),
            pltpu.VMEM((CH,), jnp.float32),
            pltpu.VMEM((LUT_SIZE,), jnp.float32),
            pltpu.VMEM((W_IN, LANES), jnp.int32),
            pltpu.VMEM((W_OUT, LANES), jnp.float32),
            pltpu.SemaphoreType.DMA,
            pltpu.SemaphoreType.DMA,
            pltpu.SemaphoreType.DMA,
            pltpu.SemaphoreType.DMA,
        ],
    )
    return kfn(flat_in, addr_t, res_t)


def kernel(tensor, addresses, results):
    flat_in = tensor.reshape(-1)
    addr_t = addresses.astype(jnp.int32).T
    res_t = results.astype(jnp.float32).T
    out = _run(flat_in, addr_t, res_t)
    return out.reshape(N, W_IN)


# speedup vs baseline: 4.0944x; 1.0626x over previous
import jax
import jax.numpy as jnp
from jax import lax
from jax.experimental import pallas as pl
from jax.experimental.pallas import tpu as pltpu, tpu_sc as plsc

N = 1048576
W_IN = 8
W_OUT = 4
P = 16

NC = 2
NS = 16
NW = NC * NS
LANES = 16

ROWS_PER_W = N // NW
CH_ROWS = 4096
CH = CH_ROWS * W_IN
NCHUNK = ROWS_PER_W // CH_ROWS
LUT_SIZE = 256 * W_IN


def _vperm(x, idx):
    return lax.gather(
        x, idx[:, None],
        lax.GatherDimensionNumbers(
            offset_dims=(), collapsed_slice_dims=(0,), start_index_map=(0,)),
        slice_sizes=(1,),
        mode=lax.GatherScatterMode.PROMISE_IN_BOUNDS)


def _body(in_hbm, addr_hbm, res_hbm, out_hbm,
          buf0, buf1, lut, addr_v, res_v,
          in_sem0, in_sem1, out_sem0, out_sem1):
    wid = lax.axis_index("s") * NC + lax.axis_index("c")
    base = wid * (ROWS_PER_W * W_IN)

    pltpu.sync_copy(addr_hbm, addr_v)
    pltpu.sync_copy(res_hbm, res_v)

    lane = lax.iota(jnp.int32, LANES)
    laneoff = lane & (W_IN - 1)
    wpat = (jnp.int32(1) << laneoff).astype(jnp.float32)
    ix4 = lane ^ 4
    ix2 = lane ^ 2
    ix1 = lane ^ 1

    nanv = jnp.full((LANES,), jnp.nan, dtype=jnp.float32)

    def init_body(k, c):
        lut[pl.ds(k * LANES, LANES)] = nanv
        return c

    lax.fori_loop(0, LUT_SIZE // LANES, init_body, 0)

    code_p = addr_v[0, :]
    for j in range(1, W_IN):
        code_p = code_p + (addr_v[j, :] << j)
    for c in range(W_OUT):
        plsc.store_scatter(lut, [code_p * W_IN + c], res_v[c, :])

    def compute(buf):
        @plsc.parallel_loop(0, CH, step=LANES, unroll=8)
        def body(off):
            v = buf[pl.ds(off, LANES)]
            t = v * wpat
            s = t + _vperm(t, ix4)
            s = s + _vperm(s, ix2)
            s = s + _vperm(s, ix1)
            idx = s.astype(jnp.int32) * W_IN + laneoff
            e = plsc.load_gather(lut, [idx])
            buf[pl.ds(off, LANES)] = jnp.where(e == e, e, v)

    bufs = (buf0, buf1)
    in_sems = (in_sem0, in_sem1)
    out_sems = (out_sem0, out_sem1)

    def in_copy(g):
        return pltpu.make_async_copy(
            in_hbm.at[pl.ds(base + g * CH, CH)], bufs[g % 2], in_sems[g % 2])

    def out_copy(g):
        return pltpu.make_async_copy(
            bufs[g % 2], out_hbm.at[pl.ds(base + g * CH, CH)], out_sems[g % 2])

    in_copy(0).start()
    for g in range(NCHUNK):
        if g + 1 < NCHUNK:
            if g >= 1:
                out_copy(g - 1).wait()
            in_copy(g + 1).start()
        in_copy(g).wait()
        compute(bufs[g % 2])
        out_copy(g).start()
    out_copy(NCHUNK - 2).wait()
    out_copy(NCHUNK - 1).wait()


@jax.jit
def _run(flat_in, addr_t, res_t):
    kfn = pl.kernel(
        _body,
        out_type=jax.ShapeDtypeStruct((N * W_IN,), jnp.float32),
        mesh=plsc.VectorSubcoreMesh(core_axis_name="c", subcore_axis_name="s"),
        compiler_params=pltpu.CompilerParams(needs_layout_passes=False),
        scratch_types=[
            pltpu.VMEM((CH,), jnp.float32),
            pltpu.VMEM((CH,), jnp.float32),
            pltpu.VMEM((LUT_SIZE,), jnp.float32),
            pltpu.VMEM((W_IN, LANES), jnp.int32),
            pltpu.VMEM((W_OUT, LANES), jnp.float32),
            pltpu.SemaphoreType.DMA,
            pltpu.SemaphoreType.DMA,
            pltpu.SemaphoreType.DMA,
            pltpu.SemaphoreType.DMA,
        ],
    )
    return kfn(flat_in, addr_t, res_t)


def kernel(tensor, addresses, results):
    flat_in = tensor.reshape(-1)
    addr_t = addresses.astype(jnp.int32).T
    res_t = results.astype(jnp.float32).T
    out = _run(flat_in, addr_t, res_t)
    return out.reshape(N, W_IN)
